# Initial kernel scaffold; baseline (speedup 1.0000x reference)
import jax
import jax.numpy as jnp
from jax.experimental import pallas as pl


def kernel(hidden_states, gate_w, gate_proj, up_proj, down_proj):
    raise NotImplementedError("write your pallas kernel here")


# speedup vs baseline: 1.6262x; 1.6262x over previous
import functools

import jax
import jax.numpy as jnp
from jax.experimental import pallas as pl
from jax.experimental.pallas import tpu as pltpu

K = 2


def _router_kernel(x_ref, gwt_ref, logits_ref, i1_ref, i2_ref, w1_ref, w2_ref):
    x = x_ref[...].astype(jnp.bfloat16)
    logits = jnp.dot(x, gwt_ref[...].astype(jnp.bfloat16),
                     preferred_element_type=jnp.float32)
    logits_ref[...] = logits
    e = logits.shape[1]
    lane = jax.lax.broadcasted_iota(jnp.int32, logits.shape, 1)
    l1 = jnp.max(logits, axis=1, keepdims=True)
    i1 = jnp.min(jnp.where(logits == l1, lane, e), axis=1, keepdims=True)
    masked = jnp.where(lane == i1, -jnp.inf, logits)
    l2 = jnp.max(masked, axis=1, keepdims=True)
    i2 = jnp.min(jnp.where(masked == l2, lane, e), axis=1, keepdims=True)
    r = jnp.exp(l2 - l1)
    w1 = 1.0 / (1.0 + r)
    i1_ref[...] = i1
    i2_ref[...] = i2
    w1_ref[...] = w1
    w2_ref[...] = 1.0 - w1


def _gmm_kernel(ft_ref, nt_ref, off_ref, cnt_ref,
                xs_ref, gp_ref, up_ref, dp_ref, y_ref, *, tm):
    e = pl.program_id(0)
    m = pl.program_id(1)

    @pl.when(m < nt_ref[e])
    def _():
        tile = ft_ref[e] + m
        start = tile * tm
        off = off_ref[e]
        cnt = cnt_ref[e]
        rows = start + jax.lax.broadcasted_iota(jnp.int32, (tm, 1), 0)
        mask = (rows >= off) & (rows < off + cnt)
        xb = xs_ref[...]
        xg = jnp.dot(xb, gp_ref[0], preferred_element_type=jnp.float32)
        xu = jnp.dot(xb, up_ref[0], preferred_element_type=jnp.float32)
        h = (xg * jax.nn.sigmoid(xg)) * xu
        y = jnp.dot(h.astype(jnp.bfloat16), dp_ref[0],
                    preferred_element_type=jnp.float32)
        first = off <= start
        prev = jnp.where(first, jnp.zeros_like(y), y_ref[...])
        y_ref[...] = jnp.where(mask, y, prev)


def kernel(hidden_states, gate_w, gate_proj, up_proj, down_proj):
    B, S, H = hidden_states.shape
    E, _, FF = gate_proj.shape
    T = B * S
    A = T * K
    x = hidden_states.reshape(T, H)

    logits, i1, i2, w1, w2 = pl.pallas_call(
        _router_kernel,
        out_shape=(
            jax.ShapeDtypeStruct((T, E), jnp.float32),
            jax.ShapeDtypeStruct((T, 1), jnp.int32),
            jax.ShapeDtypeStruct((T, 1), jnp.int32),
            jax.ShapeDtypeStruct((T, 1), jnp.float32),
            jax.ShapeDtypeStruct((T, 1), jnp.float32),
        ),
    )(x, gate_w.T)

    e_flat = jnp.concatenate([i1, i2], axis=1).reshape(A)
    perm = jnp.argsort(e_flat)
    counts = jnp.bincount(e_flat, length=E).astype(jnp.int32)
    off = (jnp.cumsum(counts) - counts).astype(jnp.int32)

    TM = 256
    NT = A // TM
    MT = T // TM + 1
    ft = off // TM
    last = off + counts - 1
    nt = jnp.where(counts > 0, last // TM - ft + 1, 0).astype(jnp.int32)
    ft = ft.astype(jnp.int32)

    xs = x.astype(jnp.bfloat16)[perm // K]
    gp = gate_proj.astype(jnp.bfloat16)
    up = up_proj.astype(jnp.bfloat16)
    dp = down_proj.astype(jnp.bfloat16)

    def x_idx(e, m, ft, nt, off, cnt):
        i = ft[e] + jnp.minimum(m, jnp.maximum(nt[e] - 1, 0))
        return (jnp.minimum(i, NT - 1), 0)

    grid_spec = pltpu.PrefetchScalarGridSpec(
        num_scalar_prefetch=4,
        grid=(E, MT),
        in_specs=[
            pl.BlockSpec((TM, H), x_idx),
            pl.BlockSpec((1, H, FF), lambda e, m, *_: (e, 0, 0)),
            pl.BlockSpec((1, H, FF), lambda e, m, *_: (e, 0, 0)),
            pl.BlockSpec((1, FF, H), lambda e, m, *_: (e, 0, 0)),
        ],
        out_specs=pl.BlockSpec((TM, H), x_idx),
    )

    y = pl.pallas_call(
        functools.partial(_gmm_kernel, tm=TM),
        grid_spec=grid_spec,
        out_shape=jax.ShapeDtypeStruct((A, H), jnp.float32),
        compiler_params=pltpu.CompilerParams(
            dimension_semantics=("arbitrary", "arbitrary")),
    )(ft, nt, off, counts, xs, gp, up, dp)

    pos = jnp.argsort(perm).reshape(T, K)
    out = y[pos[:, 0]] * w1 + y[pos[:, 1]] * w2
    return out.reshape(B, S, H), logits
